# Initial kernel scaffold; baseline (speedup 1.0000x reference)
import jax
import jax.numpy as jnp
from jax.experimental import pallas as pl


def kernel(x, edge_index, q, W1, b1, W2, b2, Ws1, bs1, Ws2, bs2):
    raise NotImplementedError("write your pallas kernel here")


# speedup vs baseline: 4.4324x; 4.4324x over previous
import jax
import jax.numpy as jnp
from jax import lax
from jax.experimental import pallas as pl
from jax.experimental.pallas import tpu as pltpu
from jax.experimental.pallas import tpu_sc as plsc

N = 10000
D = 128
H = 256
QD = 128
E = 320000

NC = 2
NS = 16
NW = NC * NS

CHUNK = 128
KPT = 79
EPAD = NW * KPT * CHUNK
NPAD = 10112
RPT = NPAD // NS

BN = 2000

_mesh = plsc.VectorSubcoreMesh(core_axis_name="c", subcore_axis_name="s")


def _gather_agg_body(tab_hbm, src_hbm, dst_hbm, z_hbm, sum_out,
                     idx_s, idx_d, rows, accum, sem):
    c = lax.axis_index("c")
    s = lax.axis_index("s")
    wid = c * NS + s
    r0 = s * RPT
    pltpu.sync_copy(z_hbm.at[pl.ds(r0, RPT)], accum.at[pl.ds(r0, RPT)])
    plsc.subcore_barrier()

    def body(j, carry):
        off = (wid * KPT + j) * CHUNK
        pltpu.sync_copy(src_hbm.at[pl.ds(off, CHUNK)], idx_s.at[0])
        pltpu.sync_copy(dst_hbm.at[pl.ds(off, CHUNK)], idx_d.at[0])
        pltpu.async_copy(tab_hbm.at[idx_s.at[0]], rows, sem).wait()
        pltpu.sync_copy(rows, accum.at[idx_d.at[0]], add=True)
        return carry

    lax.fori_loop(0, KPT, body, 0)
    plsc.subcore_barrier()
    pltpu.sync_copy(accum.at[pl.ds(r0, RPT)],
                    sum_out.at[pl.ds(c * NPAD + r0, RPT)])


def _count_body(dst_hbm, z_hbm, ones_hbm, cnt_out,
                idx_d, ones_v, accum, sem):
    c = lax.axis_index("c")
    s = lax.axis_index("s")
    wid = c * NS + s
    r0 = s * RPT
    pltpu.sync_copy(z_hbm.at[pl.ds(r0, RPT)], accum.at[pl.ds(r0, RPT)])
    pltpu.sync_copy(ones_hbm, ones_v)
    plsc.subcore_barrier()

    def body(j, carry):
        off = (wid * KPT + j) * CHUNK
        pltpu.sync_copy(dst_hbm.at[pl.ds(off, CHUNK)], idx_d.at[0])
        pltpu.sync_copy(ones_v, accum.at[idx_d.at[0]], add=True)
        return carry

    lax.fori_loop(0, KPT, body, 0)
    plsc.subcore_barrier()
    pltpu.sync_copy(accum.at[pl.ds(r0, RPT)],
                    cnt_out.at[pl.ds(c * NPAD + r0, RPT)])


_agg = pl.kernel(
    _gather_agg_body,
    out_type=jax.ShapeDtypeStruct((2 * NPAD, D), jnp.float32),
    mesh=_mesh,
    scratch_types=[pltpu.VMEM((1, CHUNK), jnp.int32),
                   pltpu.VMEM((1, CHUNK), jnp.int32),
                   pltpu.VMEM((CHUNK, D), jnp.float32),
                   pltpu.VMEM_SHARED((NPAD, D), jnp.float32),
                   pltpu.SemaphoreType.DMA],
)

_count = pl.kernel(
    _count_body,
    out_type=jax.ShapeDtypeStruct((2 * NPAD, D), jnp.float32),
    mesh=_mesh,
    scratch_types=[pltpu.VMEM((1, CHUNK), jnp.int32),
                   pltpu.VMEM((CHUNK, D), jnp.float32),
                   pltpu.VMEM_SHARED((NPAD, D), jnp.float32),
                   pltpu.SemaphoreType.DMA],
)


def _dot(a, b):
    return jnp.dot(a, b, preferred_element_type=jnp.float32)


def _layer1_tc(x_ref, s0_ref, s1_ref, c0_ref, c1_ref, q_ref, w1_ref, b1_ref,
               ha_ref, hb_ref):
    inv = 1.0 / (c0_ref[:, :1] + c1_ref[:, :1] + 1.0)
    xb = x_ref[...]
    agg = (s0_ref[...] + s1_ref[...] + xb) * inv
    qrow = _dot(q_ref[...], w1_ref[2 * D:, :]) + b1_ref[...]
    h = _dot(xb, w1_ref[:D, :]) + _dot(agg, w1_ref[D:2 * D, :]) + qrow
    h = jnp.maximum(h, 0.0)
    ha_ref[...] = h[:, :D]
    hb_ref[...] = h[:, D:]


def _layer2_tc(ha_ref, hb_ref, sa0_ref, sa1_ref, sb0_ref, sb1_ref,
               c0_ref, c1_ref, q_ref,
               w2_ref, b2_ref, ws1_ref, bs1_ref, ws2_ref, bs2_ref,
               out_ref):
    inv = 1.0 / (c0_ref[:, :1] + c1_ref[:, :1] + 1.0)
    ha = ha_ref[...]
    hb = hb_ref[...]
    aa = (sa0_ref[...] + sa1_ref[...] + ha) * inv
    ab = (sb0_ref[...] + sb1_ref[...] + hb) * inv
    qv = q_ref[...]
    qrow2 = _dot(qv, w2_ref[4 * D:, :]) + b2_ref[...]
    h = (_dot(ha, w2_ref[:D, :]) + _dot(hb, w2_ref[D:2 * D, :])
         + _dot(aa, w2_ref[2 * D:3 * D, :]) + _dot(ab, w2_ref[3 * D:4 * D, :])
         + qrow2)
    h = jnp.maximum(h, 0.0)
    qrow3 = _dot(qv, ws1_ref[H:, :]) + bs1_ref[...]
    sb = jnp.maximum(_dot(h[:, :D], ws1_ref[:D, :])
                     + _dot(h[:, D:], ws1_ref[D:H, :]) + qrow3, 0.0)
    out_ref[...] = _dot(sb, ws2_ref[...]) + bs2_ref[...]


def _row_spec(cols):
    return pl.BlockSpec((BN, cols), lambda i: (i, 0))


def _full_spec(shape):
    return pl.BlockSpec(shape, lambda i: tuple(0 for _ in shape))


_layer1_call = pl.pallas_call(
    _layer1_tc,
    grid=(N // BN,),
    in_specs=[_row_spec(D), _row_spec(D), _row_spec(D),
              _row_spec(D), _row_spec(D),
              _full_spec((1, QD)), _full_spec((2 * D + QD, H)),
              _full_spec((1, H))],
    out_specs=[_row_spec(D), _row_spec(D)],
    out_shape=[jax.ShapeDtypeStruct((N, D), jnp.float32),
               jax.ShapeDtypeStruct((N, D), jnp.float32)],
)

_layer2_call = pl.pallas_call(
    _layer2_tc,
    grid=(N // BN,),
    in_specs=[_row_spec(D), _row_spec(D),
              _row_spec(D), _row_spec(D), _row_spec(D), _row_spec(D),
              _row_spec(D), _row_spec(D),
              _full_spec((1, QD)), _full_spec((2 * H + QD, H)),
              _full_spec((1, H)), _full_spec((H + QD, D)),
              _full_spec((1, D)), _full_spec((D, 1)), _full_spec((1, 1))],
    out_specs=[pl.BlockSpec((BN, 1), lambda i: (i, 0))],
    out_shape=[jax.ShapeDtypeStruct((N, 1), jnp.float32)],
)


@jax.jit
def kernel(x, edge_index, q, W1, b1, W2, b2, Ws1, bs1, Ws2, bs2):
    pad = EPAD - E
    src = jnp.concatenate(
        [edge_index[0].astype(jnp.int32), jnp.zeros((pad,), jnp.int32)])
    dst = jnp.concatenate(
        [edge_index[1].astype(jnp.int32), jnp.full((pad,), N, jnp.int32)])
    z = jnp.zeros((NPAD, D), jnp.float32)
    ones = jnp.ones((CHUNK, D), jnp.float32)

    sums1 = _agg(x, src, dst, z)
    cnts = _count(dst, z, ones)
    c0 = cnts[:N]
    c1 = cnts[NPAD:NPAD + N]
    q2 = q.reshape(1, QD)
    ha, hb = _layer1_call(x, sums1[:N], sums1[NPAD:NPAD + N], c0, c1,
                          q2, W1, b1.reshape(1, H))

    sums2a = _agg(ha, src, dst, z)
    sums2b = _agg(hb, src, dst, z)
    (out,) = _layer2_call(ha, hb, sums2a[:N], sums2a[NPAD:NPAD + N],
                          sums2b[:N], sums2b[NPAD:NPAD + N], c0, c1,
                          q2, W2, b2.reshape(1, H), Ws1, bs1.reshape(1, D),
                          Ws2, bs2.reshape(1, 1))
    return out[:, 0]
